# baseline (speedup 1.0000x reference)
import functools

import jax
import jax.numpy as jnp
from jax import lax
from jax.experimental import pallas as pl
from jax.experimental.pallas import tpu as pltpu
from jax.experimental.pallas import tpu_sc as plsc

_BATCH = 16384
_DIM = 64
_NODES = 1000000

_info = plsc.get_sparse_core_info()
_NC = _info.num_cores
_NW = _NC * _info.num_subcores
_B_PER_W = _BATCH // _NW
_CHUNK = 128

_mesh = plsc.VectorSubcoreMesh(core_axis_name="c", subcore_axis_name="s")


@functools.partial(
    pl.kernel,
    mesh=_mesh,
    out_type=jax.ShapeDtypeStruct((_BATCH, 2 * _DIM), jnp.float32),
    scratch_types=[
        pltpu.VMEM((_B_PER_W,), jnp.int32),
        pltpu.VMEM((_B_PER_W, 2 * _DIM), jnp.float32),
        pltpu.SemaphoreType.DMA,
    ],
)
def _sc_gather(pid_hbm, pairs_hbm, rows_hbm, pid_v, rows_v, sem):
    wid = lax.axis_index("s") * _NC + lax.axis_index("c")
    base = wid * _B_PER_W
    pltpu.sync_copy(pid_hbm.at[pl.ds(base, _B_PER_W)], pid_v)
    copies = []
    for m in range(_B_PER_W // _CHUNK):
        src = pairs_hbm.at[pid_v.at[pl.ds(m * _CHUNK, _CHUNK)]]
        dst = rows_v.at[pl.ds(m * _CHUNK, _CHUNK), :]
        copies.append(pltpu.async_copy(src, dst, sem))
    for c in copies:
        c.wait()
    pltpu.sync_copy(rows_v, rows_hbm.at[pl.ds(base, _B_PER_W)])


_TC_BLK = 256


def _tc_select_body(bit_ref, rows_ref, out_ref):
    b = jnp.reshape(bit_ref[0, 0, :], (_TC_BLK, 1))
    lo = rows_ref[:, : _DIM]
    hi = rows_ref[:, _DIM :]
    out_ref[...] = jnp.where(b == 1, hi, lo)


_tc_select = pl.pallas_call(
    _tc_select_body,
    grid=(_BATCH // _TC_BLK,),
    in_specs=[
        pl.BlockSpec((1, 1, _TC_BLK), lambda i: (i, 0, 0)),
        pl.BlockSpec((_TC_BLK, 2 * _DIM), lambda i: (i, 0)),
    ],
    out_specs=pl.BlockSpec((_TC_BLK, _DIM), lambda i: (i, 0)),
    out_shape=jax.ShapeDtypeStruct((_BATCH, _DIM), jnp.float32),
)


def kernel(n_id, emb_table):
    n_id = n_id.astype(jnp.int32)
    pairs = emb_table.reshape(_NODES // 2, 2 * _DIM)
    rows = _sc_gather(lax.shift_right_logical(n_id, 1), pairs)
    bit3 = lax.bitwise_and(n_id, 1).reshape(_BATCH // _TC_BLK, 1, _TC_BLK)
    return _tc_select(bit3, rows)
